# baseline (speedup 1.0000x reference)
import functools

import jax
import jax.numpy as jnp
from jax import lax
from jax.experimental import pallas as pl
from jax.experimental.pallas import tpu as pltpu
from jax.experimental.pallas import tpu_sc as plsc

_NC = 2
_NS = 16
_NW = _NC * _NS
_NODE_WORKERS = 25

_E_ROWS = 320000 // _NW
_E_CH = 200
_N_ROWS = 10000 // _NODE_WORKERS
_N_CH = 80
_LANES = 16


def _edge_chunk_sum(buf, acc):
    def body(i, carry):
        a0, a1 = carry
        r = i * 10
        for k in range(0, 10, 2):
            a0 = a0 + buf[r + k, :]
            a1 = a1 + buf[r + k + 1, :]
        return a0, a1
    return lax.fori_loop(0, _E_CH // 10, body, acc)


def _node_chunk_sum(buf, accs):
    def body(i, carry):
        out = list(carry)
        for k in range(8):
            out[k] = out[k] + buf[i, pl.ds(k * _LANES, _LANES)]
        return tuple(out)
    return lax.fori_loop(0, _N_CH, body, accs)


def _sc_body(edge_ref, node_ref, oute_ref, outn_ref,
             ebuf0, ebuf1, nbuf0, nbuf1, stage_e, stage_n, sem0, sem1):
    cid = lax.axis_index("c")
    sid = lax.axis_index("s")
    wid = sid * _NC + cid

    base = wid * _E_ROWS
    n_chunks = _E_ROWS // _E_CH
    bufs = (ebuf0, ebuf1)
    sems = (sem0, sem1)
    copies = [None, None]
    copies[0] = pltpu.async_copy(
        edge_ref.at[pl.ds(base, _E_CH)], ebuf0, sem0)
    acc = (jnp.zeros((_LANES,), jnp.float32),
           jnp.zeros((_LANES,), jnp.float32))
    for c in range(n_chunks):
        cur = c % 2
        nxt = 1 - cur
        if c + 1 < n_chunks:
            copies[nxt] = pltpu.async_copy(
                edge_ref.at[pl.ds(base + (c + 1) * _E_CH, _E_CH)],
                bufs[nxt], sems[nxt])
        copies[cur].wait()
        acc = _edge_chunk_sum(bufs[cur], acc)
    stage_e[...] = acc[0] + acc[1]
    pltpu.sync_copy(stage_e, oute_ref.at[pl.ds(wid * _LANES, _LANES)])

    @pl.when(wid < _NODE_WORKERS)
    def _node():
        nbase = wid * _N_ROWS
        nn_chunks = _N_ROWS // _N_CH
        nbufs = (nbuf0, nbuf1)
        ncopies = [None, None]
        ncopies[0] = pltpu.async_copy(
            node_ref.at[pl.ds(nbase, _N_CH)], nbuf0, sem0)
        accs = tuple(jnp.zeros((_LANES,), jnp.float32) for _ in range(8))
        for c in range(nn_chunks):
            cur = c % 2
            nxt = 1 - cur
            if c + 1 < nn_chunks:
                ncopies[nxt] = pltpu.async_copy(
                    node_ref.at[pl.ds(nbase + (c + 1) * _N_CH, _N_CH)],
                    nbufs[nxt], sems[nxt])
            ncopies[cur].wait()
            accs = _node_chunk_sum(nbufs[cur], accs)
        for k in range(8):
            stage_n[pl.ds(k * _LANES, _LANES)] = accs[k]
        pltpu.sync_copy(stage_n, outn_ref.at[pl.ds(wid * 128, 128)])


def _sc_partials(edge_attr, node_attr):
    mesh = plsc.VectorSubcoreMesh(core_axis_name="c", subcore_axis_name="s")
    f = pl.kernel(
        _sc_body,
        out_type=[
            jax.ShapeDtypeStruct((_NW * _LANES,), jnp.float32),
            jax.ShapeDtypeStruct((_NODE_WORKERS * 128,), jnp.float32),
        ],
        mesh=mesh,
        scratch_types=[
            pltpu.VMEM((_E_CH, 16), jnp.float32),
            pltpu.VMEM((_E_CH, 16), jnp.float32),
            pltpu.VMEM((_N_CH, 128), jnp.float32),
            pltpu.VMEM((_N_CH, 128), jnp.float32),
            pltpu.VMEM((16,), jnp.float32),
            pltpu.VMEM((128,), jnp.float32),
            pltpu.SemaphoreType.DMA,
            pltpu.SemaphoreType.DMA,
        ],
    )
    return f(edge_attr, node_attr)


def _tc_body(pe_ref, pn_ref, g_ref, w1_ref, b1_ref, w2_ref, b2_ref, o_ref,
             *, inv_e, inv_n, d_edge, d_global):
    emean = jnp.sum(pe_ref[...], axis=0, keepdims=True) * inv_e
    nmean = jnp.sum(pn_ref[...], axis=0, keepdims=True) * inv_n
    wg = w1_ref[:d_global, :]
    we = w1_ref[d_global:d_global + d_edge, :]
    wn = w1_ref[d_global + d_edge:, :]
    pre = (g_ref[...] @ wg + emean @ we + nmean @ wn + b1_ref[...][None, :])
    h = jnp.maximum(pre, 0.0)
    o_ref[...] = h @ w2_ref[...] + b2_ref[...][None, :]


def kernel(node_attr, edge_index, edge_attr, global_attr, W1, b1, W2, b2):
    del edge_index
    n_edges, d_edge = edge_attr.shape
    n_nodes, d_feat = node_attr.shape
    d_global = global_attr.shape[1]
    in_features, latent = W1.shape
    out_features = W2.shape[1]

    pe, pn = _sc_partials(edge_attr, node_attr)
    pe = pe.reshape(_NW, d_edge)
    pn = pn.reshape(_NODE_WORKERS, d_feat)

    body = functools.partial(_tc_body, inv_e=1.0 / n_edges,
                             inv_n=1.0 / n_nodes, d_edge=d_edge,
                             d_global=d_global)
    out = pl.pallas_call(
        body,
        grid=(1,),
        in_specs=[
            pl.BlockSpec((_NW, d_edge), lambda i: (0, 0)),
            pl.BlockSpec((_NODE_WORKERS, d_feat), lambda i: (0, 0)),
            pl.BlockSpec((1, d_global), lambda i: (0, 0)),
            pl.BlockSpec((in_features, latent), lambda i: (0, 0)),
            pl.BlockSpec((latent,), lambda i: (0,)),
            pl.BlockSpec((latent, out_features), lambda i: (0, 0)),
            pl.BlockSpec((out_features,), lambda i: (0,)),
        ],
        out_specs=pl.BlockSpec((1, out_features), lambda i: (0, 0)),
        out_shape=jax.ShapeDtypeStruct((1, out_features), jnp.float32),
    )(pe, pn, global_attr, W1, b1, W2, b2)
    return out


# speedup vs baseline: 11.6472x; 11.6472x over previous
import functools

import jax
import jax.numpy as jnp
from jax import lax
from jax.experimental import pallas as pl
from jax.experimental.pallas import tpu as pltpu

_GRID = 10


def _body(a_ref, b_ref, g_ref, w1_ref, b1_ref, w2_ref, b2_ref,
          o_ref, acc_e, acc_n, *, grid, inv_e, inv_n, d_edge, d_global):
    i = pl.program_id(0)
    blk = a_ref.shape[1]
    ea = a_ref[...].reshape(d_edge, blk // 128, 128).sum(axis=1)
    na = jnp.sum(b_ref[...], axis=0, keepdims=True)

    @pl.when(i == 0)
    def _init():
        acc_e[...] = ea
        acc_n[0:1, :] = na

    @pl.when(i > 0)
    def _acc():
        acc_e[...] = acc_e[...] + ea
        acc_n[0:1, :] = acc_n[0:1, :] + na

    @pl.when(i == grid - 1)
    def _finish():
        s16 = jnp.sum(acc_e[...], axis=1, keepdims=True) * inv_e
        nmean = acc_n[0:1, :] * inv_n
        wg = w1_ref[:d_global, :]
        we = w1_ref[d_global:d_global + d_edge, :]
        wn = w1_ref[d_global + d_edge:, :]
        e_pre = lax.dot_general(s16, we, (((0,), (0,)), ((), ())))
        pre = (g_ref[...] @ wg + e_pre + nmean @ wn + b1_ref[...][None, :])
        h = jnp.maximum(pre, 0.0)
        o_ref[...] = h @ w2_ref[...] + b2_ref[...][None, :]


def kernel(node_attr, edge_index, edge_attr, global_attr, W1, b1, W2, b2):
    del edge_index
    n_edges, d_edge = edge_attr.shape
    n_nodes, d_feat = node_attr.shape
    d_global = global_attr.shape[1]
    in_features, latent = W1.shape
    out_features = W2.shape[1]

    et = edge_attr.T

    grid = _GRID
    blk_a = n_edges // grid
    blk_b = n_nodes // grid

    body = functools.partial(_body, grid=grid, inv_e=1.0 / n_edges,
                             inv_n=1.0 / n_nodes, d_edge=d_edge,
                             d_global=d_global)
    out = pl.pallas_call(
        body,
        grid=(grid,),
        in_specs=[
            pl.BlockSpec((d_edge, blk_a), lambda i: (0, i)),
            pl.BlockSpec((blk_b, d_feat), lambda i: (i, 0)),
            pl.BlockSpec((1, d_global), lambda i: (0, 0)),
            pl.BlockSpec((in_features, latent), lambda i: (0, 0)),
            pl.BlockSpec((latent,), lambda i: (0,)),
            pl.BlockSpec((latent, out_features), lambda i: (0, 0)),
            pl.BlockSpec((out_features,), lambda i: (0,)),
        ],
        out_specs=pl.BlockSpec((1, out_features), lambda i: (0, 0)),
        out_shape=jax.ShapeDtypeStruct((1, out_features), jnp.float32),
        scratch_shapes=[pltpu.VMEM((16, 128), jnp.float32),
                        pltpu.VMEM((8, 128), jnp.float32)],
    )(et, node_attr, global_attr, W1, b1, W2, b2)
    return out
